# speedup vs baseline: 1.0000x; 1.0000x over previous
import jax
import jax.numpy as jnp
from jax.experimental import pallas as pl


def _conv_bn(x, p, act=True):
    y = jnp.einsum('bcnk,oc->bonk', x, p["w"])
    m = jnp.mean(y, axis=(0, 2, 3), keepdims=True)
    v = jnp.var(y, axis=(0, 2, 3), keepdims=True)
    y = (y - m) / jnp.sqrt(v + 1e-6)
    y = y * p["gamma"].reshape(1, -1, 1, 1) + p["beta"].reshape(1, -1, 1, 1)
    if act:
        y = jnp.where(y >= 0, y, 0.2 * y)
    return y


def _gather_neighbour(pc, idx):
    return jax.vmap(lambda p, i: p[i])(pc, idx)


def _att_pooling(f, p):
    scores = jnp.einsum('bcnk,oc->bonk', f, p["fc_w"])
    scores = jax.nn.softmax(scores, axis=3)
    agg = jnp.sum(f * scores, axis=3, keepdims=True)
    return _conv_bn(agg, p["mlp"])


def _rel_pos(xyz, idx):
    neigh = _gather_neighbour(xyz, idx)
    tile = jnp.broadcast_to(xyz[:, :, None, :], neigh.shape)
    rel = tile - neigh
    dist = jnp.sqrt(jnp.sum(rel * rel, axis=-1, keepdims=True))
    return jnp.concatenate([dist, rel, tile, neigh], axis=-1)


def _building_block(xyz, feat, idx, p):
    f_xyz = jnp.transpose(_rel_pos(xyz, idx), (0, 3, 1, 2))
    f_xyz = _conv_bn(f_xyz, p["mlp1"])
    f_nb = jnp.transpose(_gather_neighbour(jnp.transpose(feat[..., 0], (0, 2, 1)), idx), (0, 3, 1, 2))
    f = jnp.concatenate([f_nb, f_xyz], axis=1)
    f_agg = _att_pooling(f, p["att1"])
    f_xyz = _conv_bn(f_xyz, p["mlp2"])
    f_nb = jnp.transpose(_gather_neighbour(jnp.transpose(f_agg[..., 0], (0, 2, 1)), idx), (0, 3, 1, 2))
    f = jnp.concatenate([f_nb, f_xyz], axis=1)
    return _att_pooling(f, p["att2"])


def _dilated_res_block(feat, xyz, idx, p):
    f = _conv_bn(feat, p["mlp1"])
    f = _building_block(xyz, f, idx, p["bb"])
    f = _conv_bn(f, p["mlp2"], act=False)
    sc = _conv_bn(feat, p["shortcut"], act=False)
    y = f + sc
    return jnp.where(y >= 0, y, 0.2 * y)


def _random_sample(feat, pool_idx):
    f = feat[..., 0]
    g = jax.vmap(lambda x, i: x[:, i])(f, pool_idx)
    return jnp.max(g, axis=3, keepdims=True)


def _nearest_interp(feat, idx):
    f = feat[..., 0]
    g = jax.vmap(lambda x, i: x[:, i])(f, idx[:, :, 0])
    return g[..., None]


def kernel(features, xyz0, xyz1, xyz2, xyz3, neigh_idx0, neigh_idx1, neigh_idx2, neigh_idx3, sub_idx0, sub_idx1, sub_idx2, sub_idx3, interp_idx0, interp_idx1, interp_idx2, interp_idx3, params):
    xyzs = [xyz0, xyz1, xyz2, xyz3]
    neighs = [neigh_idx0, neigh_idx1, neigh_idx2, neigh_idx3]
    subs = [sub_idx0, sub_idx1, sub_idx2, sub_idx3]
    interps = [interp_idx0, interp_idx1, interp_idx2, interp_idx3]
    feat = _conv_bn(features[..., None], params["fc0"])
    enc_list = []
    for i in range(4):
        f_enc = _dilated_res_block(feat, xyzs[i], neighs[i], params["enc"][i])
        f_samp = _random_sample(f_enc, subs[i])
        feat = f_samp
        if i == 0:
            enc_list.append(f_enc)
        enc_list.append(f_samp)
    feat = _conv_bn(enc_list[-1], params["dec0"])
    for j in range(4):
        f_int = _nearest_interp(feat, interps[3 - j])
        feat = _conv_bn(jnp.concatenate([enc_list[-j - 2], f_int], axis=1), params["dec"][j])
    feat = _conv_bn(feat, params["fc1"])
    feat = _conv_bn(feat, params["fc2"])
    out = jnp.einsum('bcnk,oc->bonk', feat, params["fc3"]["w"]) + params["fc3"]["b"].reshape(1, -1, 1, 1)
    return out[..., 0]


# speedup vs baseline: 8.2185x; 8.2183x over previous
import functools

import jax
import jax.numpy as jnp
from jax import lax
from jax.experimental import pallas as pl
from jax.experimental.pallas import tpu as pltpu
from jax.experimental.pallas import tpu_sc as plsc

_EPS = 1e-6
_K = 16


def _leaky(y):
    return jnp.where(y >= 0, y, 0.2 * y)


def _rne_bf16(x):
    u = lax.bitcast_convert_type(x, jnp.uint32)
    bias = jnp.uint32(0x7FFF) + ((u >> 16) & jnp.uint32(1))
    v = (u + bias) & jnp.uint32(0xFFFF0000)
    return lax.bitcast_convert_type(v, jnp.float32)


def _mm(x, w):
    return jnp.dot(x.astype(jnp.bfloat16), w.astype(jnp.bfloat16),
                   preferred_element_type=jnp.float32)


def _pick(m, cap=1024):
    for t in (1024, 512, 256, 128, 64, 32, 16, 8):
        if t <= cap and m % t == 0:
            return t
    return m


def _fold(p, s1, s2, count):
    w = p["w"]
    wr = _rne_bf16(w)
    mx = s1.reshape(-1) / count
    sxx = s2 / count
    def dot2(a_exact, x, contract):
        xh = x.astype(jnp.bfloat16).astype(jnp.float32)
        xl = x - xh
        return contract(a_exact, xh) + contract(a_exact, xl)

    mv = lambda a, v: jnp.sum(a * v[None, :], axis=1)
    my = dot2(wr, mx, mv)
    q = dot2(wr, sxx, lambda a, s: jnp.sum(a[:, :, None] * s[None, :, :],
                                           axis=1))
    rowdot = lambda a, qq: jnp.sum(a * qq, axis=1)
    ey2 = dot2(wr, q, lambda a, qq: rowdot(a, qq))
    var = ey2 - my * my
    scale = p["gamma"] / jnp.sqrt(var + _EPS)
    return w.T, scale, p["beta"] - my * scale


def _acc_moments(z, s1r, s2r):
    z = z.astype(jnp.bfloat16).astype(jnp.float32)

    @pl.when(pl.program_id(0) == 0)
    def _():
        s1r[...] = jnp.zeros_like(s1r)
        s2r[...] = jnp.zeros_like(s2r)

    s1r[...] += jnp.sum(z, axis=0, keepdims=True)
    s2r[...] += lax.dot_general(z, z, (((0,), (0,)), ((), ())),
                                preferred_element_type=jnp.float32,
                                precision=lax.Precision.HIGHEST)


def _affine(xs, ws, scales, off, act=True, moments=False, append=None,
            chain=None):
    m = xs[0].shape[0]
    cout = ws[0].shape[1]
    tm = _pick(m)
    grid = (m // tm,)
    nin = len(xs)

    def body(*refs):
        it = iter(refs)
        xr = [next(it) for _ in range(nin)]
        wr = [next(it) for _ in range(nin)]
        sr = [next(it) for _ in range(nin)]
        br = next(it)
        w2r = b2r = ar = None
        if chain is not None:
            w2r, b2r = next(it), next(it)
        if append is not None:
            ar = next(it)
        outr = next(it)
        y = _mm(xr[0][...], wr[0][...]) * sr[0][...]
        for j in range(1, nin):
            y = y + _mm(xr[j][...], wr[j][...]) * sr[j][...]
        y = y + br[...]
        if act:
            y = _leaky(y)
        if moments:
            s1r, s2r = next(it), next(it)
            _acc_moments(y, s1r, s2r)
        if chain is not None:
            y = _mm(y, w2r[...]) + b2r[...]
        if append is not None:
            y = jnp.concatenate([y, ar[...]], axis=1)
            npad = outr.shape[1] - y.shape[1]
            if npad:
                y = jnp.concatenate(
                    [y, jnp.zeros((y.shape[0], npad), jnp.float32)], axis=1)
        outr[...] = y

    in_specs = [pl.BlockSpec((tm, x.shape[1]), lambda i: (i, 0)) for x in xs]
    in_specs += [pl.BlockSpec(w.shape, lambda i: (0, 0)) for w in ws]
    in_specs += [pl.BlockSpec((1, cout), lambda i: (0, 0)) for _ in xs]
    in_specs += [pl.BlockSpec((1, cout), lambda i: (0, 0))]
    args = (list(xs) + list(ws) + [s.reshape(1, -1) for s in scales]
            + [off.reshape(1, -1)])
    cfin = cout
    if chain is not None:
        w2, b2 = chain
        cfin = w2.shape[1]
        in_specs += [pl.BlockSpec(w2.shape, lambda i: (0, 0)),
                     pl.BlockSpec((1, cfin), lambda i: (0, 0))]
        args += [w2, b2.reshape(1, -1)]
    if append is not None:
        in_specs += [pl.BlockSpec((tm, append.shape[1]), lambda i: (i, 0))]
        args += [append]
        cfin = -(-(cfin + append.shape[1]) // 8) * 8
    out_shapes = [jax.ShapeDtypeStruct((m, cfin), jnp.float32)]
    out_specs = [pl.BlockSpec((tm, cfin), lambda i: (i, 0))]
    if moments:
        out_shapes += [jax.ShapeDtypeStruct((1, cout), jnp.float32),
                       jax.ShapeDtypeStruct((cout, cout), jnp.float32)]
        out_specs += [pl.BlockSpec((1, cout), lambda i: (0, 0)),
                      pl.BlockSpec((cout, cout), lambda i: (0, 0))]
    res = pl.pallas_call(body, grid=grid, in_specs=in_specs,
                         out_specs=out_specs, out_shape=out_shapes)(*args)
    return res if (moments) else res[0]


def _moments(xs):
    m = xs[0].shape[0]
    c = sum(x.shape[1] for x in xs)
    tm = _pick(m)
    grid = (m // tm,)

    def body(*refs):
        xr = refs[:len(xs)]
        s1r, s2r = refs[len(xs)], refs[len(xs) + 1]
        z = xr[0][...] if len(xs) == 1 else jnp.concatenate(
            [r[...] for r in xr], axis=1)
        _acc_moments(z, s1r, s2r)

    in_specs = [pl.BlockSpec((tm, x.shape[1]), lambda i: (i, 0)) for x in xs]
    out_shapes = [jax.ShapeDtypeStruct((1, c), jnp.float32),
                  jax.ShapeDtypeStruct((c, c), jnp.float32)]
    out_specs = [pl.BlockSpec((1, c), lambda i: (0, 0)),
                 pl.BlockSpec((c, c), lambda i: (0, 0))]
    return pl.pallas_call(body, grid=grid, in_specs=in_specs,
                          out_specs=out_specs, out_shape=out_shapes)(*xs)


def _rel10(gv, xyzv, d2, p):
    pk = p * _K
    ngh = gv[:, d2:d2 + 3]
    tile = lax.broadcast_in_dim(xyzv, (p, _K, 3), (0, 2)).reshape(pk, 3)
    rel = tile - ngh
    dist = jnp.sqrt(jnp.sum(rel * rel, axis=1, keepdims=True))
    return jnp.concatenate([dist, rel, tile, ngh], axis=1)


def _relstats(g, xyz, d2):
    m = xyz.shape[0]
    p = _pick(m, cap=256)
    grid = (m // p,)
    cg = g.shape[1]

    def body(gr, xr, s1r, s2r):
        r10 = _rel10(gr[...], xr[...], d2, p)
        _acc_moments(r10, s1r, s2r)

    in_specs = [pl.BlockSpec((p * _K, cg), lambda i: (i, 0)),
                pl.BlockSpec((p, 3), lambda i: (i, 0))]
    out_shapes = [jax.ShapeDtypeStruct((1, 10), jnp.float32),
                  jax.ShapeDtypeStruct((10, 10), jnp.float32)]
    out_specs = [pl.BlockSpec((1, 10), lambda i: (0, 0)),
                 pl.BlockSpec((10, 10), lambda i: (0, 0))]
    return pl.pallas_call(body, grid=grid, in_specs=in_specs,
                          out_specs=out_specs, out_shape=out_shapes)(g, xyz)


def _att(g, xyz, fcw_t, wx1, sx1, ox1, wx2, sx2, ox2, d2, fxyz_moments):
    m = xyz.shape[0]
    d = 2 * d2
    cg = g.shape[1]
    p = _pick(m, cap=min(256, 16384 // d))
    pk = p * _K
    grid = (m // p,)
    second = wx2 is not None

    def body(*refs):
        it = iter(refs)
        gr, xr, fr, wx1r, sx1r, ox1r = (next(it) for _ in range(6))
        wx2r = sx2r = ox2r = None
        if second:
            wx2r, sx2r, ox2r = next(it), next(it), next(it)
        aggr, s1r, s2r = next(it), next(it), next(it)
        gv = gr[...]
        fnb = gv[:, :d2]
        r10 = _rel10(gv, xr[...], d2, p)
        fxyz = _leaky(_mm(r10, wx1r[...]) * sx1r[...] + ox1r[...])
        if fxyz_moments:
            xs1r, xs2r = next(it), next(it)
            _acc_moments(fxyz, xs1r, xs2r)
        if second:
            fxyz = _leaky(_mm(fxyz, wx2r[...]) * sx2r[...] + ox2r[...])
        f = jnp.concatenate([fnb, fxyz], axis=1)
        sc = _mm(f, fr[...])
        s3 = sc.reshape(p, _K, d)
        mx = jnp.max(s3, axis=1, keepdims=True)
        e = jnp.exp(s3 - mx)
        wgt = e / jnp.sum(e, axis=1, keepdims=True)
        agg = jnp.sum(f.reshape(p, _K, d) * wgt, axis=1)
        aggr[...] = agg
        _acc_moments(agg, s1r, s2r)

    in_specs = [pl.BlockSpec((pk, cg), lambda i: (i, 0)),
                pl.BlockSpec((p, 3), lambda i: (i, 0)),
                pl.BlockSpec((d, d), lambda i: (0, 0)),
                pl.BlockSpec((10, d2), lambda i: (0, 0)),
                pl.BlockSpec((1, d2), lambda i: (0, 0)),
                pl.BlockSpec((1, d2), lambda i: (0, 0))]
    args = [g, xyz, fcw_t, wx1, sx1.reshape(1, -1), ox1.reshape(1, -1)]
    if second:
        in_specs += [pl.BlockSpec((d2, d2), lambda i: (0, 0)),
                     pl.BlockSpec((1, d2), lambda i: (0, 0)),
                     pl.BlockSpec((1, d2), lambda i: (0, 0))]
        args += [wx2, sx2.reshape(1, -1), ox2.reshape(1, -1)]
    out_shapes = [jax.ShapeDtypeStruct((m, d), jnp.float32),
                  jax.ShapeDtypeStruct((1, d), jnp.float32),
                  jax.ShapeDtypeStruct((d, d), jnp.float32)]
    out_specs = [pl.BlockSpec((p, d), lambda i: (i, 0)),
                 pl.BlockSpec((1, d), lambda i: (0, 0)),
                 pl.BlockSpec((d, d), lambda i: (0, 0))]
    if fxyz_moments:
        out_shapes += [jax.ShapeDtypeStruct((1, d2), jnp.float32),
                       jax.ShapeDtypeStruct((d2, d2), jnp.float32)]
        out_specs += [pl.BlockSpec((1, d2), lambda i: (0, 0)),
                      pl.BlockSpec((d2, d2), lambda i: (0, 0))]
    return pl.pallas_call(body, grid=grid, in_specs=in_specs,
                          out_specs=out_specs, out_shape=out_shapes)(*args)


def _pool(g3):
    mp, k, c = g3.shape
    p = _pick(mp, cap=256)
    grid = (mp // p,)

    def body(gr, outr, s1r, s2r):
        y = jnp.max(gr[...], axis=1)
        outr[...] = y
        _acc_moments(y, s1r, s2r)

    in_specs = [pl.BlockSpec((p, k, c), lambda i: (i, 0, 0))]
    out_shapes = [jax.ShapeDtypeStruct((mp, c), jnp.float32),
                  jax.ShapeDtypeStruct((1, c), jnp.float32),
                  jax.ShapeDtypeStruct((c, c), jnp.float32)]
    out_specs = [pl.BlockSpec((p, c), lambda i: (i, 0)),
                 pl.BlockSpec((1, c), lambda i: (0, 0)),
                 pl.BlockSpec((c, c), lambda i: (0, 0))]
    return pl.pallas_call(body, grid=grid, in_specs=in_specs,
                          out_specs=out_specs, out_shape=out_shapes)(g3)


_NW = 32


def _sc_gather(table, idx):
    m_total = idx.shape[0]
    r, c = table.shape
    mw = m_total // _NW
    ch = _pick(mw, cap=128)
    nk = mw // ch
    mesh = plsc.VectorSubcoreMesh(core_axis_name="c", subcore_axis_name="s")

    @functools.partial(
        pl.kernel, mesh=mesh,
        out_type=jax.ShapeDtypeStruct((m_total, c), jnp.float32),
        scratch_types=[pltpu.VMEM((ch,), jnp.int32),
                       pltpu.VMEM((ch, c), jnp.float32),
                       pltpu.SemaphoreType.DMA],
        compiler_params=pltpu.CompilerParams(use_tc_tiling_on_sc=False),
    )
    def k(tbl, idxh, out, idxv, rows, sem):
        wid = lax.axis_index("s") * 2 + lax.axis_index("c")
        base = wid * mw

        def body(j, carry):
            off = base + j * ch
            pltpu.sync_copy(idxh.at[pl.ds(off, ch)], idxv)
            pltpu.async_copy(tbl.at[idxv], rows, sem).wait()
            pltpu.sync_copy(rows, out.at[pl.ds(off, ch)])
            return carry

        lax.fori_loop(0, nk, body, 0)

    return k(table, idx)


def _flat_idx(idx, tbl_rows):
    b = idx.shape[0]
    off = (jnp.arange(b, dtype=jnp.int32) * tbl_rows).reshape(b, 1, 1)
    return (idx.astype(jnp.int32) + off).reshape(-1)


def kernel(features, xyz0, xyz1, xyz2, xyz3, neigh_idx0, neigh_idx1,
           neigh_idx2, neigh_idx3, sub_idx0, sub_idx1, sub_idx2, sub_idx3,
           interp_idx0, interp_idx1, interp_idx2, interp_idx3, params):
    xyzs = [xyz0, xyz1, xyz2, xyz3]
    neighs = [neigh_idx0, neigh_idx1, neigh_idx2, neigh_idx3]
    subs = [sub_idx0, sub_idx1, sub_idx2, sub_idx3]
    interps = [interp_idx0, interp_idx1, interp_idx2, interp_idx3]

    b, cin0, n0 = features.shape
    ns = [n0] + [s.shape[1] for s in subs]
    xyz_f = [x.reshape(-1, 3) for x in xyzs]
    neigh_f = [_flat_idx(neighs[i], ns[i]) for i in range(4)]
    sub_f = [_flat_idx(subs[i], ns[i]) for i in range(4)]
    interp_f = [_flat_idx(interps[i], ns[i + 1]) for i in range(4)]

    x0 = features.transpose(0, 2, 1).reshape(b * n0, cin0)
    s1, s2 = _moments([x0])
    w, s_, o_ = _fold(params["fc0"], s1, s2, b * n0)
    feat, mf1, mf2 = _affine([x0], [w], [s_], o_, act=True, moments=True)

    enc_list = []
    for i in range(4):
        p = params["enc"][i]
        m = b * ns[i]
        mk = m * _K
        d = p["mlp1"]["w"].shape[0] * 2
        d2 = d // 2
        w1, s1_, o1_ = _fold(p["mlp1"], mf1, mf2, m)
        tbl1 = _affine([feat], [w1], [s1_], o1_, act=True, append=xyz_f[i])
        g1 = _sc_gather(tbl1, neigh_f[i])
        rs1, rs2 = _relstats(g1, xyz_f[i], d2)
        wx1, sx1, ox1 = _fold(p["bb"]["mlp1"], rs1, rs2, mk)
        agg1, a1, a2, xs1, xs2 = _att(g1, xyz_f[i], p["bb"]["att1"]["fc_w"].T,
                                      wx1, sx1, ox1, None, None, None, d2, True)
        wa1, sa1, oa1 = _fold(p["bb"]["att1"]["mlp"], a1, a2, m)
        wx2, sx2, ox2 = _fold(p["bb"]["mlp2"], xs1, xs2, mk)
        tbl2 = _affine([agg1], [wa1], [sa1], oa1, act=True, append=xyz_f[i])
        g2 = _sc_gather(tbl2, neigh_f[i])
        agg2, c1, c2 = _att(g2, xyz_f[i], p["bb"]["att2"]["fc_w"].T,
                            wx1, sx1, ox1, wx2, sx2, ox2, d2, False)
        wa2, sa2, oa2 = _fold(p["bb"]["att2"]["mlp"], c1, c2, m)
        f_out, fo1, fo2 = _affine([agg2], [wa2], [sa2], oa2, act=True,
                                  moments=True)
        wm2, sm2, om2 = _fold(p["mlp2"], fo1, fo2, m)
        wsc, ssc, osc = _fold(p["shortcut"], mf1, mf2, m)
        f_enc = _affine([f_out, feat], [wm2, wsc], [sm2, ssc], om2 + osc,
                        act=True)
        g3 = _sc_gather(f_enc, sub_f[i])
        mp = b * ns[i + 1]
        feat, mf1, mf2 = _pool(g3.reshape(mp, _K, 2 * d))
        if i == 0:
            enc_list.append(f_enc)
        enc_list.append(feat)

    wd0, sd0, od0 = _fold(params["dec0"], mf1, mf2, b * ns[4])
    dfeat = _affine([enc_list[-1]], [wd0], [sd0], od0, act=True)
    for j in range(4):
        lvl = 3 - j
        f_int = _sc_gather(dfeat, interp_f[lvl])
        skip = enc_list[-j - 2]
        mbig = skip.shape[0]
        js1, js2 = _moments([skip, f_int])
        wd, sd, od = _fold(params["dec"][j], js1, js2, mbig)
        ce = skip.shape[1]
        dfeat = _affine([skip, f_int], [wd[:ce], wd[ce:]], [sd, sd], od,
                        act=True, moments=(j == 3))
        if j == 3:
            dfeat, dm1, dm2 = dfeat

    wf1, sf1, of1 = _fold(params["fc1"], dm1, dm2, b * n0)
    h1, hm1, hm2 = _affine([dfeat], [wf1], [sf1], of1, act=True, moments=True)
    wf2, sf2, of2 = _fold(params["fc2"], hm1, hm2, b * n0)
    out = _affine([h1], [wf2], [sf2], of2, act=True,
                  chain=(params["fc3"]["w"].T, params["fc3"]["b"]))
    return out.reshape(b, n0, -1).transpose(0, 2, 1)


# speedup vs baseline: 13.0241x; 1.5847x over previous
import functools

import jax
import jax.numpy as jnp
from jax import lax
from jax.experimental import pallas as pl
from jax.experimental.pallas import tpu as pltpu
from jax.experimental.pallas import tpu_sc as plsc

_K = 16
_NW = 32


def _pick(m, cap=128):
    for t in (128, 120, 112, 104, 96, 88, 80, 72, 64, 56, 48, 40, 32, 24, 16, 8):
        if t <= cap and m % t == 0:
            return t
    return m


def _sc_gather(table, idx):
    m_total = idx.shape[0]
    r, c = table.shape
    mw = m_total // _NW
    ch = _pick(mw, cap=128)
    nk = mw // ch
    mesh = plsc.VectorSubcoreMesh(core_axis_name="c", subcore_axis_name="s")

    @functools.partial(
        pl.kernel, mesh=mesh,
        out_type=jax.ShapeDtypeStruct((m_total, c), jnp.float32),
        scratch_types=[pltpu.VMEM((ch,), jnp.int32),
                       pltpu.VMEM((ch, c), jnp.float32),
                       pltpu.SemaphoreType.DMA],
        compiler_params=pltpu.CompilerParams(use_tc_tiling_on_sc=False),
    )
    def k(tbl, idxh, out, idxv, rows, sem):
        wid = lax.axis_index("s") * 2 + lax.axis_index("c")
        base = wid * mw

        def body(j, carry):
            off = base + j * ch
            pltpu.sync_copy(idxh.at[pl.ds(off, ch)], idxv)
            pltpu.async_copy(tbl.at[idxv], rows, sem).wait()
            pltpu.sync_copy(rows, out.at[pl.ds(off, ch)])
            return carry

        lax.fori_loop(0, nk, body, 0)

    return k(table, idx)


def _flat_idx(idx, tbl_rows):
    b = idx.shape[0]
    off = (jnp.arange(b, dtype=jnp.int32) * tbl_rows).reshape(
        (b,) + (1,) * (idx.ndim - 1))
    return (idx.astype(jnp.int32) + off).reshape(-1)


def _gather_neighbour(pc, idx):
    b, n, c = pc.shape
    _, np_, k = idx.shape
    tbl = pc.reshape(b * n, c)
    cpad = -(-c // 8) * 8
    if cpad != c:
        tbl = jnp.pad(tbl, ((0, 0), (0, cpad - c)))
    rows = _sc_gather(tbl, _flat_idx(idx, n))
    if cpad != c:
        rows = rows[:, :c]
    return rows.reshape(b, np_, k, c)


def _conv_bn(x, p, act=True):
    y = jnp.einsum('bcnk,oc->bonk', x, p["w"])
    m = jnp.mean(y, axis=(0, 2, 3), keepdims=True)
    v = jnp.var(y, axis=(0, 2, 3), keepdims=True)
    y = (y - m) / jnp.sqrt(v + 1e-6)
    y = y * p["gamma"].reshape(1, -1, 1, 1) + p["beta"].reshape(1, -1, 1, 1)
    if act:
        y = jnp.where(y >= 0, y, 0.2 * y)
    return y


def _att_pooling(f, p):
    scores = jnp.einsum('bcnk,oc->bonk', f, p["fc_w"])
    scores = jax.nn.softmax(scores, axis=3)
    agg = jnp.sum(f * scores, axis=3, keepdims=True)
    return _conv_bn(agg, p["mlp"])


def _rel_pos(xyz, idx):
    neigh = _gather_neighbour(xyz, idx)
    tile = jnp.broadcast_to(xyz[:, :, None, :], neigh.shape)
    rel = tile - neigh
    dist = jnp.sqrt(jnp.sum(rel * rel, axis=-1, keepdims=True))
    return jnp.concatenate([dist, rel, tile, neigh], axis=-1)


def _building_block(xyz, feat, idx, p):
    f_xyz = jnp.transpose(_rel_pos(xyz, idx), (0, 3, 1, 2))
    f_xyz = _conv_bn(f_xyz, p["mlp1"])
    f_nb = jnp.transpose(_gather_neighbour(
        jnp.transpose(feat[..., 0], (0, 2, 1)), idx), (0, 3, 1, 2))
    f = jnp.concatenate([f_nb, f_xyz], axis=1)
    f_agg = _att_pooling(f, p["att1"])
    f_xyz = _conv_bn(f_xyz, p["mlp2"])
    f_nb = jnp.transpose(_gather_neighbour(
        jnp.transpose(f_agg[..., 0], (0, 2, 1)), idx), (0, 3, 1, 2))
    f = jnp.concatenate([f_nb, f_xyz], axis=1)
    return _att_pooling(f, p["att2"])


def _dilated_res_block(feat, xyz, idx, p):
    f = _conv_bn(feat, p["mlp1"])
    f = _building_block(xyz, f, idx, p["bb"])
    f = _conv_bn(f, p["mlp2"], act=False)
    sc = _conv_bn(feat, p["shortcut"], act=False)
    y = f + sc
    return jnp.where(y >= 0, y, 0.2 * y)


def _random_sample(feat, pool_idx):
    g = _gather_neighbour(jnp.transpose(feat[..., 0], (0, 2, 1)), pool_idx)
    return jnp.transpose(jnp.max(g, axis=2), (0, 2, 1))[..., None]


def _nearest_interp(feat, idx):
    g = _gather_neighbour(jnp.transpose(feat[..., 0], (0, 2, 1)), idx)
    return jnp.transpose(g[:, :, 0, :], (0, 2, 1))[..., None]


def kernel(features, xyz0, xyz1, xyz2, xyz3, neigh_idx0, neigh_idx1,
           neigh_idx2, neigh_idx3, sub_idx0, sub_idx1, sub_idx2, sub_idx3,
           interp_idx0, interp_idx1, interp_idx2, interp_idx3, params):
    xyzs = [xyz0, xyz1, xyz2, xyz3]
    neighs = [neigh_idx0, neigh_idx1, neigh_idx2, neigh_idx3]
    subs = [sub_idx0, sub_idx1, sub_idx2, sub_idx3]
    interps = [interp_idx0, interp_idx1, interp_idx2, interp_idx3]
    feat = _conv_bn(features[..., None], params["fc0"])
    enc_list = []
    for i in range(4):
        f_enc = _dilated_res_block(feat, xyzs[i], neighs[i], params["enc"][i])
        f_samp = _random_sample(f_enc, subs[i])
        feat = f_samp
        if i == 0:
            enc_list.append(f_enc)
        enc_list.append(f_samp)
    feat = _conv_bn(enc_list[-1], params["dec0"])
    for j in range(4):
        f_int = _nearest_interp(feat, interps[3 - j])
        feat = _conv_bn(jnp.concatenate([enc_list[-j - 2], f_int], axis=1),
                        params["dec"][j])
    feat = _conv_bn(feat, params["fc1"])
    feat = _conv_bn(feat, params["fc2"])
    out = jnp.einsum('bcnk,oc->bonk', feat, params["fc3"]["w"]) \
        + params["fc3"]["b"].reshape(1, -1, 1, 1)
    return out[..., 0]
